# Initial kernel scaffold; baseline (speedup 1.0000x reference)
import jax
import jax.numpy as jnp
from jax.experimental import pallas as pl


def kernel(seq, node_s, edge_index, edge_s, batch, embed_w, pn_w, pn_b, pe_w, pe_b, Wq0, bq0, Wk0, bk0, Wv0, bv0, We0, Ws0, bs0, Wq1, bq1, Wk1, bk1, Wv1, bv1, We1, Ws1, bs1, Wq2, bq2, Wk2, bk2, Wv2, bv2, We2, Ws2, bs2):
    raise NotImplementedError("write your pallas kernel here")


# speedup vs baseline: 2.6605x; 2.6605x over previous
import functools
import math

import jax
import jax.numpy as jnp
from jax import lax
from jax.experimental import pallas as pl
from jax.experimental.pallas import tpu as pltpu
from jax.experimental.pallas import tpu_sc as plsc

N = 10000
E = 160000
NG = 16
NC = 2
NS = 16
NW = NC * NS

BN = 1000
BE = 1280
G = 40


def _lrelu(t):
    return jnp.where(t >= 0, t, 0.01 * t)


def _node_init_body(seq_ref, ns_ref, emb_ref, pnw_ref, pnb_ref, o_ref):
    s = seq_ref[0, 0, :]
    onehot = (s[:, None] == lax.broadcasted_iota(jnp.int32, (1, 21), 1)
              ).astype(jnp.float32)
    emb = jnp.dot(onehot, emb_ref[...], preferred_element_type=jnp.float32)
    xin = jnp.concatenate([emb, ns_ref[...]], axis=1)
    o_ref[...] = (jnp.dot(xin, pnw_ref[...], preferred_element_type=jnp.float32)
                  + pnb_ref[0, :])


def _node_init(seq3, node_s, embed_w, pn_w, pn_b2):
    BQ = 400
    return pl.pallas_call(
        _node_init_body,
        grid=(N // BQ,),
        in_specs=[
            pl.BlockSpec((1, 1, BQ), lambda i: (i, 0, 0)),
            pl.BlockSpec((BQ, 6), lambda i: (i, 0)),
            pl.BlockSpec((21, 20), lambda i: (0, 0)),
            pl.BlockSpec((26, 128), lambda i: (0, 0)),
            pl.BlockSpec((1, 128), lambda i: (0, 0)),
        ],
        out_specs=pl.BlockSpec((BQ, 128), lambda i: (i, 0)),
        out_shape=jax.ShapeDtypeStruct((N, 128), jnp.float32),
    )(seq3, node_s, embed_w, pn_w, pn_b2)


def _mm_bias_body(x_ref, w_ref, b_ref, o_ref):
    o_ref[...] = (jnp.dot(x_ref[...], w_ref[...],
                          preferred_element_type=jnp.float32) + b_ref[0, :])


def _edge_proj(edge_s, pe_w, pe_b2):
    BEP = 2000
    return pl.pallas_call(
        _mm_bias_body,
        grid=(E // BEP,),
        in_specs=[
            pl.BlockSpec((BEP, 39), lambda i: (i, 0)),
            pl.BlockSpec((39, 128), lambda i: (0, 0)),
            pl.BlockSpec((1, 128), lambda i: (0, 0)),
        ],
        out_specs=pl.BlockSpec((BEP, 128), lambda i: (i, 0)),
        out_shape=jax.ShapeDtypeStruct((E, 128), jnp.float32),
    )(edge_s, pe_w, pe_b2)


def _proj_body(x_ref, w_ref, b_ref, q_ref, k_ref, v_ref, s_ref):
    y = (jnp.dot(x_ref[...], w_ref[...], preferred_element_type=jnp.float32)
         + b_ref[0, :])
    C = q_ref.shape[1]
    q_ref[...] = y[:, :C]
    k_ref[...] = y[:, C:2 * C]
    v_ref[...] = y[:, 2 * C:3 * C]
    s_ref[...] = y[:, 3 * C:]


def _proj(x, w4, b4, C):
    din = x.shape[1]
    outs = [jax.ShapeDtypeStruct((N, C), jnp.float32)] * 4
    return pl.pallas_call(
        _proj_body,
        grid=(N // BN,),
        in_specs=[
            pl.BlockSpec((BN, din), lambda i: (i, 0)),
            pl.BlockSpec((din, 4 * C), lambda i: (0, 0)),
            pl.BlockSpec((1, 4 * C), lambda i: (0, 0)),
        ],
        out_specs=[pl.BlockSpec((BN, C), lambda i: (i, 0))] * 4,
        out_shape=outs,
    )(x, w4, b4)


def _ee_body(e_ref, w_ref, o_ref):
    o_ref[...] = jnp.dot(e_ref[...], w_ref[...],
                         preferred_element_type=jnp.float32)


def _ee_proj(e, We, C):
    BEP = 2000
    return pl.pallas_call(
        _ee_body,
        grid=(E // BEP,),
        in_specs=[
            pl.BlockSpec((BEP, 128), lambda i: (i, 0)),
            pl.BlockSpec((128, C), lambda i: (0, 0)),
        ],
        out_specs=pl.BlockSpec((BEP, C), lambda i: (i, 0)),
        out_shape=jax.ShapeDtypeStruct((E, C), jnp.float32),
    )(e, We)


def _alpha_body(qd_ref, ks_ref, ee_ref, a_ref, m_ref, *, inv_sqrt_c):
    al = jnp.sum(qd_ref[...] * (ks_ref[...] + ee_ref[...]), axis=1) * inv_sqrt_c
    a_ref[...] = al.reshape(1, 1, -1)
    bm = jnp.max(al).reshape(1, 1)

    @pl.when(pl.program_id(0) == 0)
    def _():
        m_ref[...] = bm

    @pl.when(pl.program_id(0) != 0)
    def _():
        m_ref[...] = jnp.maximum(m_ref[...], bm)


def _alpha(qd, ks, ee, C):
    body = functools.partial(_alpha_body, inv_sqrt_c=1.0 / math.sqrt(C))
    return pl.pallas_call(
        body,
        grid=(E // BE,),
        in_specs=[
            pl.BlockSpec((BE, C), lambda i: (i, 0)),
            pl.BlockSpec((BE, C), lambda i: (i, 0)),
            pl.BlockSpec((BE, C), lambda i: (i, 0)),
        ],
        out_specs=[
            pl.BlockSpec((1, 1, BE), lambda i: (i, 0, 0)),
            pl.BlockSpec((1, 1), lambda i: (0, 0)),
        ],
        out_shape=[
            jax.ShapeDtypeStruct((E // BE, 1, BE), jnp.float32),
            jax.ShapeDtypeStruct((1, 1), jnp.float32),
        ],
    )(qd, ks, ee)


_MW = 112


def _pass_cols(C):
    return [(p * _MW, min(_MW, C - p * _MW)) for p in range((C + _MW - 1) // _MW)]


def _w_body(a_ref, m_ref, vs_ref, ee_ref, *out_refs):
    ex = jnp.exp(a_ref[0, 0, :] - m_ref[...][0, 0])
    w = (vs_ref[...] + ee_ref[...]) * ex[:, None]
    C = vs_ref.shape[1]
    exb = jnp.broadcast_to(ex[:, None], (ex.shape[0], 16))
    for r, (c0, used) in zip(out_refs, _pass_cols(C)):
        cols = [w[:, c0:c0 + used]]
        if used < _MW:
            cols.append(jnp.zeros((w.shape[0], _MW - used), jnp.float32))
        cols.append(exb)
        r[...] = jnp.concatenate(cols, axis=1)


def _w_stage(a3, gmax, vs, ee, C):
    ng = len(_pass_cols(C))
    return pl.pallas_call(
        _w_body,
        grid=(E // BE,),
        in_specs=[
            pl.BlockSpec((1, 1, BE), lambda i: (i, 0, 0)),
            pl.BlockSpec((1, 1), lambda i: (0, 0)),
            pl.BlockSpec((BE, C), lambda i: (i, 0)),
            pl.BlockSpec((BE, C), lambda i: (i, 0)),
        ],
        out_specs=[pl.BlockSpec((BE, 128), lambda i: (i, 0))] * ng,
        out_shape=[jax.ShapeDtypeStruct((E, 128), jnp.float32)] * ng,
    )(a3, gmax, vs, ee)


def _combine(wabs, xs, C):
    ng = len(wabs)
    pcols = _pass_cols(C)

    def body(*refs):
        part_refs = refs[:ng]
        xs_ref, o_ref = refs[ng:]
        den = part_refs[0][0][:, _MW:_MW + 1]
        msg = jnp.concatenate(
            [r[0][:, :used] for r, (_, used) in zip(part_refs, pcols)], axis=1)
        o_ref[...] = _lrelu(msg / jnp.maximum(den, 1e-16) + xs_ref[...])

    wspec = pl.BlockSpec((1, BN, 128), lambda i: (i // 5, i % 5, 0))
    return pl.pallas_call(
        body,
        grid=(N // BN,),
        in_specs=([wspec] * ng + [pl.BlockSpec((BN, C), lambda i: (i, 0))]),
        out_specs=pl.BlockSpec((BN, C), lambda i: (i, 0)),
        out_shape=jax.ShapeDtypeStruct((N, C), jnp.float32),
    )(*wabs, xs)


def _pool_body(b_ref, x_ref, o_ref, acc_ref, cnt_ref):
    @pl.when(pl.program_id(0) == 0)
    def _():
        acc_ref[...] = jnp.zeros_like(acc_ref)
        cnt_ref[...] = jnp.zeros_like(cnt_ref)

    b = b_ref[0, 0, :]
    onehot = (b[:, None] == lax.broadcasted_iota(jnp.int32, (1, NG), 1)
              ).astype(jnp.float32)
    acc_ref[...] += lax.dot_general(onehot, x_ref[...],
                                    (((0,), (0,)), ((), ())),
                                    preferred_element_type=jnp.float32)
    cnt_ref[...] += jnp.broadcast_to(jnp.sum(onehot, axis=0)[:, None],
                                     cnt_ref.shape)
    o_ref[...] = acc_ref[...] / jnp.maximum(cnt_ref[...], 1.0)


def _pool(batch3, x, C):
    BQ = 400
    return pl.pallas_call(
        _pool_body,
        grid=(N // BQ,),
        in_specs=[
            pl.BlockSpec((1, 1, BQ), lambda i: (i, 0, 0)),
            pl.BlockSpec((BQ, C), lambda i: (i, 0)),
        ],
        out_specs=pl.BlockSpec((NG, C), lambda i: (0, 0)),
        out_shape=jax.ShapeDtypeStruct((NG, C), jnp.float32),
        scratch_shapes=[
            pltpu.VMEM((NG, C), jnp.float32),
            pltpu.VMEM((NG, C), jnp.float32),
        ],
    )(batch3, x)


@functools.cache
def _make_gather(C):
    mesh = plsc.VectorSubcoreMesh(core_axis_name="c", subcore_axis_name="s",
                                  num_cores=NC, num_subcores=NS)
    per_w = E // NW
    steps = per_w // G

    def body(dst_hbm, src_hbm, q_hbm, k_hbm, v_hbm, qd_hbm, ks_hbm, vs_hbm,
             idxd, idxs, qbuf, kbuf, vbuf, sd, ss, sq, sk, sv):
        wid = lax.axis_index("s") * NC + lax.axis_index("c")
        base = wid * per_w

        @pl.loop(0, steps)
        def _(g):
            row0 = base + g * G
            cd = pltpu.async_copy(dst_hbm.at[pl.ds(row0, G)], idxd, sd)
            cs = pltpu.async_copy(src_hbm.at[pl.ds(row0, G)], idxs, ss)
            cd.wait()
            cs.wait()
            cq = pltpu.async_copy(q_hbm.at[idxd], qbuf, sq)
            ck = pltpu.async_copy(k_hbm.at[idxs], kbuf, sk)
            cv = pltpu.async_copy(v_hbm.at[idxs], vbuf, sv)
            cq.wait()
            ck.wait()
            cv.wait()
            pltpu.sync_copy(qbuf, qd_hbm.at[pl.ds(row0, G)])
            pltpu.sync_copy(kbuf, ks_hbm.at[pl.ds(row0, G)])
            pltpu.sync_copy(vbuf, vs_hbm.at[pl.ds(row0, G)])

    out_type = [jax.ShapeDtypeStruct((E, C), jnp.float32)] * 3
    scratch = [
        pltpu.VMEM((G,), jnp.int32),
        pltpu.VMEM((G,), jnp.int32),
        pltpu.VMEM((G, C), jnp.float32),
        pltpu.VMEM((G, C), jnp.float32),
        pltpu.VMEM((G, C), jnp.float32),
        pltpu.SemaphoreType.DMA,
        pltpu.SemaphoreType.DMA,
        pltpu.SemaphoreType.DMA,
        pltpu.SemaphoreType.DMA,
        pltpu.SemaphoreType.DMA,
    ]
    return pl.kernel(body, out_type=out_type, mesh=mesh, scratch_types=scratch)


NH = N // NC
NACC = 5120
RB = NACC // NS
TRASH = 5100
GS = 80


@functools.cache
def _make_scatter():
    mesh = plsc.VectorSubcoreMesh(core_axis_name="c", subcore_axis_name="s",
                                  num_cores=NC, num_subcores=NS)
    per_w = E // NS
    steps = per_w // GS

    def body(dst_hbm, w_hbm, za_hbm, outab, idx, wbuf, stg, acc):
        c = lax.axis_index("c")
        s = lax.axis_index("s")
        r0 = s * RB
        pltpu.sync_copy(za_hbm, stg)
        pltpu.sync_copy(stg, acc.at[pl.ds(r0, RB)])
        plsc.subcore_barrier()

        @pl.loop(0, steps)
        def _(g):
            row0 = s * per_w + g * GS
            pltpu.sync_copy(dst_hbm.at[pl.ds(row0, GS)], idx)
            pltpu.sync_copy(w_hbm.at[pl.ds(row0, GS)], wbuf)
            for t in range(GS // 16):
                sl = pl.ds(t * 16, 16)
                v = idx[sl] - c * NH
                ok = (v >= 0) & (v < NH)
                idx[sl] = jnp.where(ok, v, TRASH)
            pltpu.sync_copy(wbuf, acc.at[idx], add=True)

        plsc.subcore_barrier()
        pltpu.sync_copy(acc.at[pl.ds(r0, RB)], stg)
        pltpu.sync_copy(stg, outab.at[c, pl.ds(r0, RB)])

    out_type = jax.ShapeDtypeStruct((2, NACC, 128), jnp.float32)
    scratch = [
        pltpu.VMEM((GS,), jnp.int32),
        pltpu.VMEM((GS, 128), jnp.float32),
        pltpu.VMEM((RB, 128), jnp.float32),
        pltpu.VMEM_SHARED((NACC, 128), jnp.float32),
    ]
    return pl.kernel(body, out_type=out_type, mesh=mesh, scratch_types=scratch)


def _layer(x, e, dst, src, Wq, bq, Wk, bk, Wv, bv, We, Ws, bs):
    C = Wq.shape[1]
    w4 = jnp.concatenate([Wq, Wk, Wv, Ws], axis=1)
    b4 = jnp.concatenate([bq, bk, bv, bs]).reshape(1, -1)
    q, k, v, xs = _proj(x, w4, b4, C)
    ee = _ee_proj(e, We, C)
    qd, ks, vs = _make_gather(C)(dst, src, q, k, v)
    a3, gmax = _alpha(qd, ks, ee, C)
    ws = _w_stage(a3, gmax, vs, ee, C)
    za = jnp.zeros((RB, 128), jnp.float32)
    scat = _make_scatter()
    wabs = [scat(dst, w, za) for w in ws]
    return _combine(wabs, xs, C)


def kernel(seq, node_s, edge_index, edge_s, batch, embed_w, pn_w, pn_b, pe_w,
           pe_b, Wq0, bq0, Wk0, bk0, Wv0, bv0, We0, Ws0, bs0,
           Wq1, bq1, Wk1, bk1, Wv1, bv1, We1, Ws1, bs1,
           Wq2, bq2, Wk2, bk2, Wv2, bv2, We2, Ws2, bs2):
    src = edge_index[0]
    dst = edge_index[1]
    seq3 = seq.reshape(25, 1, 400)
    batch3 = batch.reshape(25, 1, 400)
    x = _node_init(seq3, node_s, embed_w, pn_w, pn_b.reshape(1, -1))
    e = _edge_proj(edge_s, pe_w, pe_b.reshape(1, -1))
    x = _layer(x, e, dst, src, Wq0, bq0, Wk0, bk0, Wv0, bv0, We0, Ws0, bs0)
    x = _layer(x, e, dst, src, Wq1, bq1, Wk1, bk1, Wv1, bv1, We1, Ws1, bs1)
    x = _layer(x, e, dst, src, Wq2, bq2, Wk2, bk2, Wv2, bv2, We2, Ws2, bs2)
    return _pool(batch3, x, 256)
